# baseline (speedup 1.0000x reference)
import functools

import jax
import jax.numpy as jnp
from jax import lax
from jax.experimental import pallas as pl
from jax.experimental.pallas import tpu as pltpu
from jax.experimental.pallas import tpu_sc as plsc

NC = 2
NS = 16
NW = NC * NS


@functools.lru_cache(maxsize=None)
def _make_gather(n: int, vocab: int, dim: int):
    rows_per_w = n // NW
    chunk = 16
    nbuf = 4
    nchunk = rows_per_w // chunk
    mesh = plsc.VectorSubcoreMesh(core_axis_name="c", subcore_axis_name="s")

    @functools.partial(
        pl.kernel,
        mesh=mesh,
        out_type=jax.ShapeDtypeStruct((n, dim), jnp.float32),
        scratch_types=[
            pltpu.VMEM((nchunk, chunk), jnp.int32),
            pltpu.VMEM((nbuf, chunk, dim), jnp.float32),
        ]
        + [pltpu.SemaphoreType.DMA] * (2 * nbuf),
    )
    def gather(ids_hbm, table_hbm, out_hbm, idx_v, rows_v, *sems):
        gsem = sems[:nbuf]
        ssem = sems[nbuf:]
        wid = lax.axis_index("s") * NC + lax.axis_index("c")
        base = wid * rows_per_w
        pltpu.sync_copy(ids_hbm.at[wid], idx_v)

        def start_gather(c):
            return pltpu.async_copy(
                table_hbm.at[idx_v.at[c]],
                rows_v.at[c % nbuf], gsem[c % nbuf])

        pending_gather = [start_gather(b) for b in range(min(nbuf, nchunk))]
        pending_gather += [None] * (nbuf - len(pending_gather))
        pending_store = [None] * nbuf
        for c in range(nchunk):
            g = c - 1 + nbuf
            if c >= 1 and g < nchunk:
                pb = (c - 1) % nbuf
                pending_store[pb].wait()
                pending_store[pb] = None
                pending_gather[pb] = start_gather(g)
            buf = c % nbuf
            pending_gather[buf].wait()
            pending_store[buf] = pltpu.async_copy(
                rows_v.at[buf], out_hbm.at[pl.ds(base + c * chunk, chunk)],
                ssem[buf])
        for st in pending_store:
            if st is not None:
                st.wait()

    def call(flat_ids, table):
        return gather(flat_ids.reshape(NW, nchunk, chunk), table)

    return call


def kernel(input_ids, embed_table):
    b, s = input_ids.shape
    vocab, dim = embed_table.shape
    n = b * s
    flat = input_ids.reshape(n).astype(jnp.int32)
    out = _make_gather(n, vocab, dim)(flat, embed_table)
    return out.reshape(b, s, dim)


# speedup vs baseline: 1.0133x; 1.0133x over previous
import functools

import jax
import jax.numpy as jnp
from jax import lax
from jax.experimental import pallas as pl
from jax.experimental.pallas import tpu as pltpu
from jax.experimental.pallas import tpu_sc as plsc

NC = 2
NS = 16
NW = NC * NS


@functools.lru_cache(maxsize=None)
def _make_gather(batch: int, seq: int, vocab: int, dim: int):
    n = batch * seq
    rows_per_w = n // NW
    w_per_batch = seq // rows_per_w
    chunk = 32
    nbuf = 3
    nchunk = rows_per_w // chunk
    mesh = plsc.VectorSubcoreMesh(core_axis_name="c", subcore_axis_name="s")

    @functools.partial(
        pl.kernel,
        mesh=mesh,
        out_type=jax.ShapeDtypeStruct((batch, seq, dim), jnp.float32),
        scratch_types=[
            pltpu.VMEM((rows_per_w,), jnp.int32),
            pltpu.VMEM((nbuf, chunk, dim), jnp.float32),
        ]
        + [pltpu.SemaphoreType.DMA] * (2 * nbuf),
    )
    def gather(ids_hbm, table_hbm, out_hbm, idx_v, rows_v, *sems):
        gsem = sems[:nbuf]
        ssem = sems[nbuf:]
        wid = lax.axis_index("s") * NC + lax.axis_index("c")
        bi = wid // w_per_batch
        boff = (wid % w_per_batch) * rows_per_w
        pltpu.sync_copy(ids_hbm.at[bi, pl.ds(boff, rows_per_w)], idx_v)

        def start_gather(c):
            return pltpu.async_copy(
                table_hbm.at[idx_v.at[pl.ds(c * chunk, chunk)]],
                rows_v.at[c % nbuf], gsem[c % nbuf])

        pending_gather = [start_gather(b) for b in range(min(nbuf, nchunk))]
        pending_gather += [None] * (nbuf - len(pending_gather))
        pending_store = [None] * nbuf
        for c in range(nchunk):
            g = c - 1 + nbuf
            if c >= 1 and g < nchunk:
                pb = (c - 1) % nbuf
                pending_store[pb].wait()
                pending_store[pb] = None
                pending_gather[pb] = start_gather(g)
            buf = c % nbuf
            pending_gather[buf].wait()
            pending_store[buf] = pltpu.async_copy(
                rows_v.at[buf],
                out_hbm.at[bi, pl.ds(boff + c * chunk, chunk)],
                ssem[buf])
        for st in pending_store:
            if st is not None:
                st.wait()

    return gather


def kernel(input_ids, embed_table):
    batch, seq = input_ids.shape
    vocab, dim = embed_table.shape
    return _make_gather(batch, seq, vocab, dim)(
        input_ids.astype(jnp.int32), embed_table)


# speedup vs baseline: 1.0212x; 1.0078x over previous
import functools

import jax
import jax.numpy as jnp
from jax import lax
from jax.experimental import pallas as pl
from jax.experimental.pallas import tpu as pltpu
from jax.experimental.pallas import tpu_sc as plsc

NC = 2
NS = 16
NW = NC * NS


@functools.lru_cache(maxsize=None)
def _make_gather(batch: int, seq: int, vocab: int, dim: int):
    n = batch * seq
    rows_per_w = n // NW
    w_per_batch = seq // rows_per_w
    big = 40
    nbig = rows_per_w // big
    chunks = [big] * nbig + ([rows_per_w - big * nbig] if rows_per_w % big else [])
    offs = [big * i for i in range(len(chunks))]
    nbuf = 3
    nchunk = len(chunks)
    mesh = plsc.VectorSubcoreMesh(core_axis_name="c", subcore_axis_name="s")

    @functools.partial(
        pl.kernel,
        mesh=mesh,
        out_type=jax.ShapeDtypeStruct((batch, seq, dim), jnp.float32),
        scratch_types=[
            pltpu.VMEM((rows_per_w,), jnp.int32),
            pltpu.VMEM((nbuf, big, dim), jnp.float32),
        ]
        + [pltpu.SemaphoreType.DMA] * (2 * nbuf),
    )
    def gather(ids_hbm, table_hbm, out_hbm, idx_v, rows_v, *sems):
        gsem = sems[:nbuf]
        ssem = sems[nbuf:]
        wid = lax.axis_index("s") * NC + lax.axis_index("c")
        bi = wid // w_per_batch
        boff = (wid % w_per_batch) * rows_per_w
        pltpu.sync_copy(ids_hbm.at[bi, pl.ds(boff, rows_per_w)], idx_v)

        def start_gather(c):
            return pltpu.async_copy(
                table_hbm.at[idx_v.at[pl.ds(offs[c], chunks[c])]],
                rows_v.at[c % nbuf, pl.ds(0, chunks[c])], gsem[c % nbuf])

        pending_gather = [start_gather(b) for b in range(min(nbuf, nchunk))]
        pending_gather += [None] * (nbuf - len(pending_gather))
        pending_store = [None] * nbuf
        for c in range(nchunk):
            g = c - 1 + nbuf
            if c >= 1 and g < nchunk:
                pb = (c - 1) % nbuf
                pending_store[pb].wait()
                pending_store[pb] = None
                pending_gather[pb] = start_gather(g)
            buf = c % nbuf
            pending_gather[buf].wait()
            pending_store[buf] = pltpu.async_copy(
                rows_v.at[buf, pl.ds(0, chunks[c])],
                out_hbm.at[bi, pl.ds(boff + offs[c], chunks[c])],
                ssem[buf])
        for st in pending_store:
            if st is not None:
                st.wait()

    return gather


def kernel(input_ids, embed_table):
    batch, seq = input_ids.shape
    vocab, dim = embed_table.shape
    return _make_gather(batch, seq, vocab, dim)(
        input_ids.astype(jnp.int32), embed_table)
